# baseline (speedup 1.0000x reference)
import jax
import jax.numpy as jnp
from jax import lax
from jax.experimental import pallas as pl
from jax.experimental.pallas import tpu as pltpu
from jax.experimental.pallas import tpu_sc as plsc

NUM_CORES = 2
NUM_SUBCORES = 16
NUM_WORKERS = NUM_CORES * NUM_SUBCORES
LANES = 16

B = 16384
F = 26
D = 32
NUM_FEATURES = 1000000
BPW = B // NUM_WORKERS
ROWS_PER_CHUNK = 16
CHUNKS = BPW // ROWS_PER_CHUNK
IDX_PER_CHUNK = ROWS_PER_CHUNK * F
RING = 4


def _fm_body(x_hbm, w0_hbm, bias_hbm, embt_hbm, out_hbm,
             xv, w0v, bv, ebs, outv, esem, bsem):
  wid = lax.axis_index("s") * NUM_CORES + lax.axis_index("c")
  pltpu.sync_copy(
      x_hbm.at[pl.ds(pl.multiple_of(wid * BPW * F, 8), BPW * F)], xv)
  pltpu.sync_copy(w0_hbm, w0v)
  bias_cp = pltpu.make_async_copy(bias_hbm.at[xv], bv, bsem)
  bias_cp.start()

  def emb_copies(c, slot):
    off = pl.multiple_of(c * IDX_PER_CHUNK, 8)
    idx = xv.at[pl.ds(off, IDX_PER_CHUNK)]
    for d in range(D):
      yield pltpu.make_async_copy(
          embt_hbm.at[d].at[idx], ebs.at[slot, d], esem.at[slot])

  lane = lax.iota(jnp.int32, LANES)
  fidx = [F * lane + f for f in range(F)]

  def compute(c, slot):
    res = jnp.zeros((LANES,), jnp.float32)
    for d in range(D):
      acc = jnp.zeros((LANES,), jnp.float32)
      sq = jnp.zeros((LANES,), jnp.float32)
      for f in range(F):
        v = plsc.load_gather(ebs.at[slot, d], [fidx[f]])
        acc = acc + v
        sq = sq + v * v
      res = res + (acc * acc - sq)
    res = 0.5 * res
    bbase = c * IDX_PER_CHUNK
    for f in range(F):
      res = res + plsc.load_gather(bv, [bbase + fidx[f]])
    outv[pl.ds(c * ROWS_PER_CHUNK, LANES)] = res + w0v[...]

  for c in range(RING):
    for cp in emb_copies(c, c):
      cp.start()
  bias_cp.wait()

  @pl.loop(0, CHUNKS)
  def _chunk_loop(c):
    slot = lax.rem(c, RING)
    for cp in emb_copies(c, slot):
      cp.wait()
    compute(c, slot)
    nxt = c + RING

    @pl.when(nxt < CHUNKS)
    def _():
      for cp in emb_copies(nxt, slot):
        cp.start()

  pltpu.sync_copy(outv,
                  out_hbm.at[pl.ds(pl.multiple_of(wid * BPW, BPW), BPW)])


@jax.jit
def _fm_call(x_flat, w016, bias_flat, emb_t):
  return pl.kernel(
      _fm_body,
      out_type=jax.ShapeDtypeStruct((B,), jnp.float32),
      mesh=plsc.VectorSubcoreMesh(core_axis_name="c", subcore_axis_name="s"),
      compiler_params=pltpu.CompilerParams(
          needs_layout_passes=False, use_tc_tiling_on_sc=False),
      scratch_types=[
          pltpu.VMEM((BPW * F,), jnp.int32),
          pltpu.VMEM((LANES,), jnp.float32),
          pltpu.VMEM((BPW * F,), jnp.float32),
          pltpu.VMEM((RING, D, IDX_PER_CHUNK), jnp.float32),
          pltpu.VMEM((BPW,), jnp.float32),
          pltpu.SemaphoreType.DMA((RING,)),
          pltpu.SemaphoreType.DMA,
      ],
  )(x_flat, w016, bias_flat, emb_t)


def kernel(x, w0, bias_table, emb_table):
  x = x.astype(jnp.int32)
  w016 = jnp.broadcast_to(w0.astype(jnp.float32), (LANES,))
  return _fm_call(x.reshape(-1), w016, bias_table.reshape(-1), emb_table.T)


# speedup vs baseline: 4.1687x; 4.1687x over previous
import jax
import jax.numpy as jnp
from jax import lax
from jax.experimental import pallas as pl
from jax.experimental.pallas import tpu as pltpu
from jax.experimental.pallas import tpu_sc as plsc

NUM_CORES = 2
NUM_SUBCORES = 16
NUM_WORKERS = NUM_CORES * NUM_SUBCORES
LANES = 16

B = 16384
F = 26
D = 32
NUM_FEATURES = 1000000
WIDE = 128
ROWS_PER_WIDE = WIDE // D
BPW = B // NUM_WORKERS
ROWS_PER_CHUNK = 8
CHUNKS = BPW // ROWS_PER_CHUNK
IDX_PER_CHUNK = ROWS_PER_CHUNK * F
RING = 3


def _fm_body(x_hbm, x4_hbm, w0_hbm, bias_hbm, emb_hbm, out_hbm,
             xv, x4v, w0v, bv, ebs, outv, esem, bsem):
  wid = lax.axis_index("s") * NUM_CORES + lax.axis_index("c")
  pltpu.sync_copy(
      x_hbm.at[pl.ds(pl.multiple_of(wid * BPW * F, 8), BPW * F)], xv)
  pltpu.sync_copy(
      x4_hbm.at[pl.ds(pl.multiple_of(wid * BPW * F, 8), BPW * F)], x4v)
  pltpu.sync_copy(w0_hbm, w0v)
  bias_cp = pltpu.make_async_copy(bias_hbm.at[xv], bv, bsem)
  bias_cp.start()

  def emb_copy(c, slot):
    off = pl.multiple_of(c * IDX_PER_CHUNK, 8)
    return pltpu.make_async_copy(
        emb_hbm.at[x4v.at[pl.ds(off, IDX_PER_CHUNK)]], ebs.at[slot],
        esem.at[slot])

  lane = lax.iota(jnp.int32, LANES)
  row_mask = lane < ROWS_PER_CHUNK
  posv = [F * lane + f for f in range(F)]

  def compute(c, slot):
    cbase = c * IDX_PER_CHUNK
    eb = ebs.at[slot]
    cols = []
    for f in range(F):
      idx = plsc.load_gather(xv, [cbase + posv[f]], mask=row_mask)
      cols.append((idx & (ROWS_PER_WIDE - 1)) * D)
    res = jnp.zeros((LANES,), jnp.float32)
    for d in range(D):
      acc = jnp.zeros((LANES,), jnp.float32)
      sq = jnp.zeros((LANES,), jnp.float32)
      for f in range(F):
        v = plsc.load_gather(eb, [posv[f], cols[f] + d], mask=row_mask)
        acc = acc + v
        sq = sq + v * v
      res = res + (acc * acc - sq)
    res = 0.5 * res
    for f in range(F):
      res = res + plsc.load_gather(bv, [cbase + posv[f]], mask=row_mask)
    plsc.store_compressed(outv.at[pl.ds(c * ROWS_PER_CHUNK, LANES)],
                          res + w0v[...], mask=row_mask)

  for c in range(RING):
    emb_copy(c, c).start()
  bias_cp.wait()

  @pl.loop(0, CHUNKS)
  def _chunk_loop(c):
    slot = lax.rem(c, RING)
    emb_copy(c, slot).wait()
    compute(c, slot)
    nxt = c + RING

    @pl.when(nxt < CHUNKS)
    def _():
      emb_copy(nxt, slot).start()

  pltpu.sync_copy(outv.at[pl.ds(0, BPW)],
                  out_hbm.at[pl.ds(pl.multiple_of(wid * BPW, BPW), BPW)])


@jax.jit
def _fm_call(x_flat, x4_flat, w016, bias_flat, emb_wide):
  return pl.kernel(
      _fm_body,
      out_type=jax.ShapeDtypeStruct((B,), jnp.float32),
      mesh=plsc.VectorSubcoreMesh(core_axis_name="c", subcore_axis_name="s"),
      compiler_params=pltpu.CompilerParams(
          needs_layout_passes=False, use_tc_tiling_on_sc=False),
      scratch_types=[
          pltpu.VMEM((BPW * F,), jnp.int32),
          pltpu.VMEM((BPW * F,), jnp.int32),
          pltpu.VMEM((LANES,), jnp.float32),
          pltpu.VMEM((BPW * F,), jnp.float32),
          pltpu.VMEM((RING, IDX_PER_CHUNK, WIDE), jnp.float32),
          pltpu.VMEM((BPW + ROWS_PER_CHUNK,), jnp.float32),
          pltpu.SemaphoreType.DMA((RING,)),
          pltpu.SemaphoreType.DMA,
      ],
  )(x_flat, x4_flat, w016, bias_flat, emb_wide)


def kernel(x, w0, bias_table, emb_table):
  x = x.astype(jnp.int32)
  w016 = jnp.broadcast_to(w0.astype(jnp.float32), (LANES,))
  return _fm_call(x.reshape(-1), (x >> 2).reshape(-1), w016,
                  bias_table.reshape(-1),
                  emb_table.reshape(NUM_FEATURES * D // WIDE, WIDE))
